# baseline (device time: 29284 ns/iter reference)
import jax
import jax.numpy as jnp
from jax import lax
from jax.experimental import pallas as pl
from jax.experimental.pallas import tpu as pltpu

N_DEV = 32

B = 2
SQ = 128
SKV = 128
DH = 64


def kernel(x, Wq, K_ext, V_ext, Wo):
    hq_per = K_ext.shape[2]
    cols = hq_per * DH
    d_model = x.shape[-1]
    rows = B * SQ
    ch = rows // N_DEV

    idx = lax.axis_index("i")
    Wq_s = lax.dynamic_slice_in_dim(Wq, idx * cols, cols, axis=1).astype(
        jnp.bfloat16
    )
    Wo_s = lax.dynamic_slice_in_dim(Wo, idx * cols, cols, axis=0).astype(
        jnp.bfloat16
    )
    x_bf = x.astype(jnp.bfloat16)
    K_t = jnp.transpose(K_ext, (0, 2, 1, 3)).astype(jnp.bfloat16)
    V_t = jnp.transpose(V_ext, (0, 2, 1, 3)).astype(jnp.bfloat16)

    def body(x_ref, wq_ref, k_ref, v_ref, wo_ref, out_ref,
             ctx_ref, accum_ref, recv1_ref, gather_ref,
             send1_sems, recv1_sems, send2_sems, recv2_sems):
        my = lax.axis_index("i")

        barrier = pltpu.get_barrier_semaphore()
        for j in range(1, N_DEV):
            tgt = (my + j) % N_DEV
            pl.semaphore_signal(
                barrier, inc=1,
                device_id=(tgt,), device_id_type=pl.DeviceIdType.MESH,
            )
        pl.semaphore_wait(barrier, N_DEV - 1)

        x2 = x_ref[:].reshape(rows, d_model)
        q_all = jnp.dot(
            x2, wq_ref[:], preferred_element_type=jnp.float32
        ).astype(jnp.bfloat16)

        ii = lax.broadcasted_iota(jnp.int32, (SQ, SKV), 0) // 64
        jj = lax.broadcasted_iota(jnp.int32, (SQ, SKV), 1) // 64
        mask = (ii == jj) | (jj == 0) | ((ii + jj) % 3 == 0)

        for b in range(B):
            for h in range(hq_per):
                q = q_all[b * SQ:(b + 1) * SQ, h * DH:(h + 1) * DH]
                kk = k_ref[b, h]
                s = lax.dot_general(
                    q, kk, (((1,), (1,)), ((), ())),
                    preferred_element_type=jnp.float32,
                ) * 0.125
                s = jnp.where(mask, s, -1e9)
                m = jnp.max(s, axis=1, keepdims=True)
                w = jnp.exp(s - m)
                w = (w / jnp.sum(w, axis=1, keepdims=True)).astype(jnp.bfloat16)
                ctx_ref[b * SQ:(b + 1) * SQ, h * DH:(h + 1) * DH] = jnp.dot(
                    w, v_ref[b, h], preferred_element_type=jnp.float32
                ).astype(jnp.bfloat16)

        accum_ref[:] = jnp.dot(
            ctx_ref[:], wo_ref[:], preferred_element_type=jnp.float32
        ).astype(jnp.bfloat16)

        for j in range(1, N_DEV):
            tgt = (my + j) % N_DEV
            rdma = pltpu.make_async_remote_copy(
                src_ref=accum_ref.at[pl.ds(tgt * ch, ch)],
                dst_ref=recv1_ref.at[my],
                send_sem=send1_sems.at[tgt],
                recv_sem=recv1_sems.at[my],
                device_id=(tgt,),
                device_id_type=pl.DeviceIdType.MESH,
            )
            rdma.start()

        recv1_ref[pl.ds(my, 1)] = accum_ref[pl.ds(my * ch, ch)].reshape(
            1, ch, d_model
        )

        for j in range(1, N_DEV):
            src = (my + j) % N_DEV
            pltpu.make_async_remote_copy(
                src_ref=accum_ref.at[pl.ds(0, ch)],
                dst_ref=recv1_ref.at[src],
                send_sem=send1_sems.at[0],
                recv_sem=recv1_sems.at[src],
                device_id=(src,),
                device_id_type=pl.DeviceIdType.MESH,
            ).wait_recv()

        red = jnp.sum(
            recv1_ref[:].astype(jnp.float32), axis=0
        ).astype(jnp.bfloat16)
        gather_ref[pl.ds(my * ch, ch)] = red

        for j in range(1, N_DEV):
            tgt = (my + j) % N_DEV
            rdma = pltpu.make_async_remote_copy(
                src_ref=gather_ref.at[pl.ds(my * ch, ch)],
                dst_ref=gather_ref.at[pl.ds(my * ch, ch)],
                send_sem=send2_sems.at[tgt],
                recv_sem=recv2_sems.at[my],
                device_id=(tgt,),
                device_id_type=pl.DeviceIdType.MESH,
            )
            rdma.start()

        for j in range(1, N_DEV):
            src = (my + j) % N_DEV
            pltpu.make_async_remote_copy(
                src_ref=gather_ref.at[pl.ds(0, ch)],
                dst_ref=gather_ref.at[pl.ds(src * ch, ch)],
                send_sem=send2_sems.at[0],
                recv_sem=recv2_sems.at[src],
                device_id=(src,),
                device_id_type=pl.DeviceIdType.MESH,
            ).wait_recv()

        out_ref[:] = gather_ref[:].astype(jnp.float32).reshape(B, SQ, d_model)

        for j in range(1, N_DEV):
            tgt = (my + j) % N_DEV
            pltpu.make_async_remote_copy(
                src_ref=accum_ref.at[pl.ds(tgt * ch, ch)],
                dst_ref=recv1_ref.at[my],
                send_sem=send1_sems.at[tgt],
                recv_sem=recv1_sems.at[my],
                device_id=(tgt,),
                device_id_type=pl.DeviceIdType.MESH,
            ).wait_send()
            pltpu.make_async_remote_copy(
                src_ref=gather_ref.at[pl.ds(my * ch, ch)],
                dst_ref=gather_ref.at[pl.ds(my * ch, ch)],
                send_sem=send2_sems.at[tgt],
                recv_sem=recv2_sems.at[my],
                device_id=(tgt,),
                device_id_type=pl.DeviceIdType.MESH,
            ).wait_send()

    return pl.pallas_call(
        body,
        out_shape=jax.ShapeDtypeStruct((B, SQ, d_model), jnp.float32),
        in_specs=[
            pl.BlockSpec(memory_space=pltpu.VMEM),
            pl.BlockSpec(memory_space=pltpu.VMEM),
            pl.BlockSpec(memory_space=pltpu.VMEM),
            pl.BlockSpec(memory_space=pltpu.VMEM),
            pl.BlockSpec(memory_space=pltpu.VMEM),
        ],
        out_specs=pl.BlockSpec(memory_space=pltpu.VMEM),
        scratch_shapes=[
            pltpu.VMEM((rows, cols), jnp.bfloat16),
            pltpu.VMEM((rows, d_model), jnp.bfloat16),
            pltpu.VMEM((N_DEV, ch, d_model), jnp.bfloat16),
            pltpu.VMEM((rows, d_model), jnp.bfloat16),
            pltpu.SemaphoreType.DMA((N_DEV,)),
            pltpu.SemaphoreType.DMA((N_DEV,)),
            pltpu.SemaphoreType.DMA((N_DEV,)),
            pltpu.SemaphoreType.DMA((N_DEV,)),
        ],
        compiler_params=pltpu.CompilerParams(collective_id=0),
    )(x_bf, Wq_s, K_t, V_t, Wo_s)


# device time: 29053 ns/iter; 1.0080x vs baseline; 1.0080x over previous
import jax
import jax.numpy as jnp
from jax import lax
from jax.experimental import pallas as pl
from jax.experimental.pallas import tpu as pltpu

N_DEV = 32

B = 2
SQ = 128
SKV = 128
DH = 64


def kernel(x, Wq, K_ext, V_ext, Wo):
    hq_per = K_ext.shape[2]
    cols = hq_per * DH
    d_model = x.shape[-1]
    rows = B * SQ
    ch = rows // N_DEV

    idx = lax.axis_index("i")
    Wq_s = lax.dynamic_slice_in_dim(Wq, idx * cols, cols, axis=1).astype(
        jnp.bfloat16
    )
    Wo_s = lax.dynamic_slice_in_dim(Wo, idx * cols, cols, axis=0).astype(
        jnp.bfloat16
    )
    K_t = jnp.transpose(K_ext, (0, 2, 1, 3)).astype(jnp.bfloat16)
    V_t = jnp.transpose(V_ext, (0, 2, 1, 3)).astype(jnp.bfloat16)

    def body(x_ref, wq_ref, k_ref, v_ref, wo_ref, out_ref,
             ctx_ref, accum_ref, recv1_ref, gather_ref,
             send1_sems, recv1_sems, send2_sems, recv2_sems):
        my = lax.axis_index("i")

        barrier = pltpu.get_barrier_semaphore()
        for j in range(1, N_DEV):
            tgt = (my + j) % N_DEV
            pl.semaphore_signal(
                barrier, inc=1,
                device_id=(tgt,), device_id_type=pl.DeviceIdType.MESH,
            )
        pl.semaphore_wait(barrier, N_DEV - 1)

        x2 = x_ref[:].reshape(rows, d_model).astype(jnp.bfloat16)
        q_all = jnp.dot(
            x2, wq_ref[:], preferred_element_type=jnp.float32
        ).astype(jnp.bfloat16)

        ii = lax.broadcasted_iota(jnp.int32, (SQ, SKV), 0) // 64
        jj = lax.broadcasted_iota(jnp.int32, (SQ, SKV), 1) // 64
        mask = (ii == jj) | (jj == 0) | ((ii + jj) % 3 == 0)

        for b in range(B):
            for h in range(hq_per):
                q = q_all[b * SQ:(b + 1) * SQ, h * DH:(h + 1) * DH]
                kk = k_ref[b, h]
                s = lax.dot_general(
                    q, kk, (((1,), (1,)), ((), ())),
                    preferred_element_type=jnp.float32,
                ) * 0.125
                s = jnp.where(mask, s, -1e9)
                m = jnp.max(s, axis=1, keepdims=True)
                w = jnp.exp(s - m)
                w = (w / jnp.sum(w, axis=1, keepdims=True)).astype(jnp.bfloat16)
                ctx_ref[b * SQ:(b + 1) * SQ, h * DH:(h + 1) * DH] = jnp.dot(
                    w, v_ref[b, h], preferred_element_type=jnp.float32
                ).astype(jnp.bfloat16)

        blk_rows = 64
        for blk in range(rows // blk_rows):
            accum_ref[blk * blk_rows:(blk + 1) * blk_rows, :] = jnp.dot(
                ctx_ref[blk * blk_rows:(blk + 1) * blk_rows, :], wo_ref[:],
                preferred_element_type=jnp.float32,
            ).astype(jnp.bfloat16)
            for jj in range(blk_rows // ch):
                c = blk * (blk_rows // ch) + (my + jj) % (blk_rows // ch)
                rdma = pltpu.make_async_remote_copy(
                    src_ref=accum_ref.at[pl.ds(c * ch, ch)],
                    dst_ref=recv1_ref.at[my],
                    send_sem=send1_sems.at[c],
                    recv_sem=recv1_sems.at[my],
                    device_id=(c,),
                    device_id_type=pl.DeviceIdType.MESH,
                )

                @pl.when(c != my)
                def _():
                    rdma.start()

        recv1_ref[pl.ds(my, 1)] = accum_ref[pl.ds(my * ch, ch)].reshape(
            1, ch, d_model
        )

        for j in range(1, N_DEV):
            src = (my + j) % N_DEV
            pltpu.make_async_remote_copy(
                src_ref=accum_ref.at[pl.ds(0, ch)],
                dst_ref=recv1_ref.at[src],
                send_sem=send1_sems.at[0],
                recv_sem=recv1_sems.at[src],
                device_id=(src,),
                device_id_type=pl.DeviceIdType.MESH,
            ).wait_recv()

        red = jnp.sum(
            recv1_ref[:].astype(jnp.float32), axis=0
        ).astype(jnp.bfloat16)
        gather_ref[pl.ds(my * ch, ch)] = red

        for j in range(1, N_DEV):
            tgt = (my + j) % N_DEV
            rdma = pltpu.make_async_remote_copy(
                src_ref=gather_ref.at[pl.ds(my * ch, ch)],
                dst_ref=gather_ref.at[pl.ds(my * ch, ch)],
                send_sem=send2_sems.at[tgt],
                recv_sem=recv2_sems.at[my],
                device_id=(tgt,),
                device_id_type=pl.DeviceIdType.MESH,
            )
            rdma.start()

        for j in range(1, N_DEV):
            src = (my + j) % N_DEV
            pltpu.make_async_remote_copy(
                src_ref=gather_ref.at[pl.ds(0, ch)],
                dst_ref=gather_ref.at[pl.ds(src * ch, ch)],
                send_sem=send2_sems.at[0],
                recv_sem=recv2_sems.at[src],
                device_id=(src,),
                device_id_type=pl.DeviceIdType.MESH,
            ).wait_recv()

        out_ref[:] = gather_ref[:].astype(jnp.float32).reshape(B, SQ, d_model)

        for j in range(1, N_DEV):
            tgt = (my + j) % N_DEV
            pltpu.make_async_remote_copy(
                src_ref=accum_ref.at[pl.ds(tgt * ch, ch)],
                dst_ref=recv1_ref.at[my],
                send_sem=send1_sems.at[tgt],
                recv_sem=recv1_sems.at[my],
                device_id=(tgt,),
                device_id_type=pl.DeviceIdType.MESH,
            ).wait_send()
            pltpu.make_async_remote_copy(
                src_ref=gather_ref.at[pl.ds(my * ch, ch)],
                dst_ref=gather_ref.at[pl.ds(my * ch, ch)],
                send_sem=send2_sems.at[tgt],
                recv_sem=recv2_sems.at[my],
                device_id=(tgt,),
                device_id_type=pl.DeviceIdType.MESH,
            ).wait_send()

    return pl.pallas_call(
        body,
        out_shape=jax.ShapeDtypeStruct((B, SQ, d_model), jnp.float32),
        in_specs=[
            pl.BlockSpec(memory_space=pltpu.VMEM),
            pl.BlockSpec(memory_space=pltpu.VMEM),
            pl.BlockSpec(memory_space=pltpu.VMEM),
            pl.BlockSpec(memory_space=pltpu.VMEM),
            pl.BlockSpec(memory_space=pltpu.VMEM),
        ],
        out_specs=pl.BlockSpec(memory_space=pltpu.VMEM),
        scratch_shapes=[
            pltpu.VMEM((rows, cols), jnp.bfloat16),
            pltpu.VMEM((rows, d_model), jnp.bfloat16),
            pltpu.VMEM((N_DEV, ch, d_model), jnp.bfloat16),
            pltpu.VMEM((rows, d_model), jnp.bfloat16),
            pltpu.SemaphoreType.DMA((N_DEV,)),
            pltpu.SemaphoreType.DMA((N_DEV,)),
            pltpu.SemaphoreType.DMA((N_DEV,)),
            pltpu.SemaphoreType.DMA((N_DEV,)),
        ],
        compiler_params=pltpu.CompilerParams(collective_id=0),
    )(x, Wq_s, K_t, V_t, Wo_s)


# device time: 29047 ns/iter; 1.0082x vs baseline; 1.0002x over previous
import jax
import jax.numpy as jnp
from jax import lax
from jax.experimental import pallas as pl
from jax.experimental.pallas import tpu as pltpu

N_DEV = 32

B = 2
SQ = 128
SKV = 128
DH = 64


def kernel(x, Wq, K_ext, V_ext, Wo):
    hq_per = K_ext.shape[2]
    cols = hq_per * DH
    d_model = x.shape[-1]
    rows = B * SQ
    ch = rows // N_DEV

    idx = lax.axis_index("i")
    Wq_s = lax.dynamic_slice_in_dim(Wq, idx * cols, cols, axis=1).astype(
        jnp.bfloat16
    )
    Wo_s = lax.dynamic_slice_in_dim(Wo, idx * cols, cols, axis=0).astype(
        jnp.bfloat16
    )
    K_t = K_ext.reshape(B, SKV, hq_per * DH)
    V_t = V_ext.reshape(B, SKV, hq_per * DH)

    def body(x_ref, wq_ref, k_ref, v_ref, wo_ref, out_ref,
             ctx_ref, accum_ref, recv1_ref, gather_ref,
             send1_sems, recv1_sems, send2_sems, recv2_sems):
        my = lax.axis_index("i")

        barrier = pltpu.get_barrier_semaphore()
        for j in range(1, N_DEV):
            tgt = (my + j) % N_DEV
            pl.semaphore_signal(
                barrier, inc=1,
                device_id=(tgt,), device_id_type=pl.DeviceIdType.MESH,
            )
        pl.semaphore_wait(barrier, N_DEV - 1)

        x2 = x_ref[:].reshape(rows, d_model).astype(jnp.bfloat16)
        q_all = jnp.dot(
            x2, wq_ref[:], preferred_element_type=jnp.float32
        ).astype(jnp.bfloat16)

        ii = lax.broadcasted_iota(jnp.int32, (SQ, SKV), 0) // 64
        jj = lax.broadcasted_iota(jnp.int32, (SQ, SKV), 1) // 64
        mask = (ii == jj) | (jj == 0) | ((ii + jj) % 3 == 0)

        for b in range(B):
            for h in range(hq_per):
                q = q_all[b * SQ:(b + 1) * SQ, h * DH:(h + 1) * DH]
                kk = k_ref[b, :, h * DH:(h + 1) * DH].astype(jnp.bfloat16)
                s = lax.dot_general(
                    q, kk, (((1,), (1,)), ((), ())),
                    preferred_element_type=jnp.float32,
                ) * 0.125
                s = jnp.where(mask, s, -1e9)
                m = jnp.max(s, axis=1, keepdims=True)
                w = jnp.exp(s - m)
                w = (w / jnp.sum(w, axis=1, keepdims=True)).astype(jnp.bfloat16)
                ctx_ref[b * SQ:(b + 1) * SQ, h * DH:(h + 1) * DH] = jnp.dot(
                    w, v_ref[b, :, h * DH:(h + 1) * DH].astype(jnp.bfloat16),
                    preferred_element_type=jnp.float32,
                ).astype(jnp.bfloat16)

        blk_rows = 64
        for blk in range(rows // blk_rows):
            accum_ref[blk * blk_rows:(blk + 1) * blk_rows, :] = jnp.dot(
                ctx_ref[blk * blk_rows:(blk + 1) * blk_rows, :], wo_ref[:],
                preferred_element_type=jnp.float32,
            ).astype(jnp.bfloat16)
            for jj in range(blk_rows // ch):
                c = blk * (blk_rows // ch) + (my + jj) % (blk_rows // ch)
                rdma = pltpu.make_async_remote_copy(
                    src_ref=accum_ref.at[pl.ds(c * ch, ch)],
                    dst_ref=recv1_ref.at[my],
                    send_sem=send1_sems.at[c],
                    recv_sem=recv1_sems.at[my],
                    device_id=(c,),
                    device_id_type=pl.DeviceIdType.MESH,
                )

                @pl.when(c != my)
                def _():
                    rdma.start()

        recv1_ref[pl.ds(my, 1)] = accum_ref[pl.ds(my * ch, ch)].reshape(
            1, ch, d_model
        )

        for j in range(1, N_DEV):
            src = (my + j) % N_DEV
            pltpu.make_async_remote_copy(
                src_ref=accum_ref.at[pl.ds(0, ch)],
                dst_ref=recv1_ref.at[src],
                send_sem=send1_sems.at[0],
                recv_sem=recv1_sems.at[src],
                device_id=(src,),
                device_id_type=pl.DeviceIdType.MESH,
            ).wait_recv()

        red = jnp.sum(
            recv1_ref[:].astype(jnp.float32), axis=0
        ).astype(jnp.bfloat16)
        gather_ref[pl.ds(my * ch, ch)] = red

        for j in range(1, N_DEV):
            tgt = (my + j) % N_DEV
            rdma = pltpu.make_async_remote_copy(
                src_ref=gather_ref.at[pl.ds(my * ch, ch)],
                dst_ref=gather_ref.at[pl.ds(my * ch, ch)],
                send_sem=send2_sems.at[tgt],
                recv_sem=recv2_sems.at[my],
                device_id=(tgt,),
                device_id_type=pl.DeviceIdType.MESH,
            )
            rdma.start()

        for j in range(1, N_DEV):
            src = (my + j) % N_DEV
            pltpu.make_async_remote_copy(
                src_ref=gather_ref.at[pl.ds(0, ch)],
                dst_ref=gather_ref.at[pl.ds(src * ch, ch)],
                send_sem=send2_sems.at[0],
                recv_sem=recv2_sems.at[src],
                device_id=(src,),
                device_id_type=pl.DeviceIdType.MESH,
            ).wait_recv()

        out_ref[:] = gather_ref[:].astype(jnp.float32).reshape(B, SQ, d_model)

        for j in range(1, N_DEV):
            tgt = (my + j) % N_DEV
            pltpu.make_async_remote_copy(
                src_ref=accum_ref.at[pl.ds(tgt * ch, ch)],
                dst_ref=recv1_ref.at[my],
                send_sem=send1_sems.at[tgt],
                recv_sem=recv1_sems.at[my],
                device_id=(tgt,),
                device_id_type=pl.DeviceIdType.MESH,
            ).wait_send()
            pltpu.make_async_remote_copy(
                src_ref=gather_ref.at[pl.ds(my * ch, ch)],
                dst_ref=gather_ref.at[pl.ds(my * ch, ch)],
                send_sem=send2_sems.at[tgt],
                recv_sem=recv2_sems.at[my],
                device_id=(tgt,),
                device_id_type=pl.DeviceIdType.MESH,
            ).wait_send()

    return pl.pallas_call(
        body,
        out_shape=jax.ShapeDtypeStruct((B, SQ, d_model), jnp.float32),
        in_specs=[
            pl.BlockSpec(memory_space=pltpu.VMEM),
            pl.BlockSpec(memory_space=pltpu.VMEM),
            pl.BlockSpec(memory_space=pltpu.VMEM),
            pl.BlockSpec(memory_space=pltpu.VMEM),
            pl.BlockSpec(memory_space=pltpu.VMEM),
        ],
        out_specs=pl.BlockSpec(memory_space=pltpu.VMEM),
        scratch_shapes=[
            pltpu.VMEM((rows, cols), jnp.bfloat16),
            pltpu.VMEM((rows, d_model), jnp.bfloat16),
            pltpu.VMEM((N_DEV, ch, d_model), jnp.bfloat16),
            pltpu.VMEM((rows, d_model), jnp.bfloat16),
            pltpu.SemaphoreType.DMA((N_DEV,)),
            pltpu.SemaphoreType.DMA((N_DEV,)),
            pltpu.SemaphoreType.DMA((N_DEV,)),
            pltpu.SemaphoreType.DMA((N_DEV,)),
        ],
        compiler_params=pltpu.CompilerParams(collective_id=0),
    )(x, Wq_s, K_t, V_t, Wo_s)


# device time: 27630 ns/iter; 1.0599x vs baseline; 1.0513x over previous
import jax
import jax.numpy as jnp
from jax import lax
from jax.experimental import pallas as pl
from jax.experimental.pallas import tpu as pltpu

N_DEV = 32

B = 2
SQ = 128
SKV = 128
DH = 64


def kernel(x, Wq, K_ext, V_ext, Wo):
    hq_per = K_ext.shape[2]
    cols = hq_per * DH
    d_model = x.shape[-1]
    rows = B * SQ
    ch = rows // N_DEV

    idx = lax.axis_index("i")
    Wq_s = lax.dynamic_slice_in_dim(Wq, idx * cols, cols, axis=1).astype(
        jnp.bfloat16
    )
    Wo_s = lax.dynamic_slice_in_dim(Wo, idx * cols, cols, axis=0).astype(
        jnp.bfloat16
    )
    K_t = K_ext.reshape(B, SKV, hq_per * DH)
    V_t = V_ext.reshape(B, SKV, hq_per * DH)

    def body(x_ref, wq_ref, k_ref, v_ref, wo_ref, out_ref,
             ctx_ref, accum_ref, recv1_ref, gather_ref,
             send1_sems, recv1_sems, send2_sems, recv2_sems):
        my = lax.axis_index("i")

        barrier = pltpu.get_barrier_semaphore()
        for j in range(1, N_DEV):
            tgt = (my + j) % N_DEV
            pl.semaphore_signal(
                barrier, inc=1,
                device_id=(tgt,), device_id_type=pl.DeviceIdType.MESH,
            )
        pl.semaphore_wait(barrier, N_DEV - 1)

        x2 = x_ref[:].reshape(rows, d_model).astype(jnp.bfloat16)
        q_all = jnp.dot(
            x2, wq_ref[:], preferred_element_type=jnp.float32
        ).astype(jnp.bfloat16)

        pairs = [(b, h) for b in range(B) for h in range(hq_per)]
        s_all = jnp.concatenate(
            [
                lax.dot_general(
                    q_all[b * SQ:(b + 1) * SQ, h * DH:(h + 1) * DH],
                    k_ref[b, :, h * DH:(h + 1) * DH].astype(jnp.bfloat16),
                    (((1,), (1,)), ((), ())),
                    preferred_element_type=jnp.float32,
                )
                for (b, h) in pairs
            ],
            axis=0,
        ) * 0.125
        n_stack = len(pairs) * SQ
        ii = (lax.broadcasted_iota(jnp.int32, (n_stack, SKV), 0) % SQ) // 64
        jj = lax.broadcasted_iota(jnp.int32, (n_stack, SKV), 1) // 64
        mask = (ii == jj) | (jj == 0) | ((ii + jj) % 3 == 0)
        s_all = jnp.where(mask, s_all, -1e9)
        m = jnp.max(s_all, axis=1, keepdims=True)
        w_all = jnp.exp(s_all - m)
        w_all = (w_all / jnp.sum(w_all, axis=1, keepdims=True)).astype(
            jnp.bfloat16
        )
        for p, (b, h) in enumerate(pairs):
            ctx_ref[b * SQ:(b + 1) * SQ, h * DH:(h + 1) * DH] = jnp.dot(
                w_all[p * SQ:(p + 1) * SQ, :],
                v_ref[b, :, h * DH:(h + 1) * DH].astype(jnp.bfloat16),
                preferred_element_type=jnp.float32,
            ).astype(jnp.bfloat16)

        blk_rows = 64
        for blk in range(rows // blk_rows):
            accum_ref[blk * blk_rows:(blk + 1) * blk_rows, :] = jnp.dot(
                ctx_ref[blk * blk_rows:(blk + 1) * blk_rows, :], wo_ref[:],
                preferred_element_type=jnp.float32,
            ).astype(jnp.bfloat16)
            for jj in range(blk_rows // ch):
                c = blk * (blk_rows // ch) + (my + jj) % (blk_rows // ch)
                rdma = pltpu.make_async_remote_copy(
                    src_ref=accum_ref.at[pl.ds(c * ch, ch)],
                    dst_ref=recv1_ref.at[my],
                    send_sem=send1_sems.at[c],
                    recv_sem=recv1_sems.at[my],
                    device_id=(c,),
                    device_id_type=pl.DeviceIdType.MESH,
                )

                @pl.when(c != my)
                def _():
                    rdma.start()

        recv1_ref[pl.ds(my, 1)] = accum_ref[pl.ds(my * ch, ch)].reshape(
            1, ch, d_model
        )

        for j in range(1, N_DEV):
            src = (my + j) % N_DEV
            pltpu.make_async_remote_copy(
                src_ref=accum_ref.at[pl.ds(0, ch)],
                dst_ref=recv1_ref.at[src],
                send_sem=send1_sems.at[0],
                recv_sem=recv1_sems.at[src],
                device_id=(src,),
                device_id_type=pl.DeviceIdType.MESH,
            ).wait_recv()

        red = jnp.sum(
            recv1_ref[:].astype(jnp.float32), axis=0
        ).astype(jnp.bfloat16)
        gather_ref[pl.ds(my * ch, ch)] = red

        for j in range(1, N_DEV):
            tgt = (my + j) % N_DEV
            rdma = pltpu.make_async_remote_copy(
                src_ref=gather_ref.at[pl.ds(my * ch, ch)],
                dst_ref=gather_ref.at[pl.ds(my * ch, ch)],
                send_sem=send2_sems.at[tgt],
                recv_sem=recv2_sems.at[my],
                device_id=(tgt,),
                device_id_type=pl.DeviceIdType.MESH,
            )
            rdma.start()

        for j in range(1, N_DEV):
            src = (my + j) % N_DEV
            pltpu.make_async_remote_copy(
                src_ref=gather_ref.at[pl.ds(0, ch)],
                dst_ref=gather_ref.at[pl.ds(src * ch, ch)],
                send_sem=send2_sems.at[0],
                recv_sem=recv2_sems.at[src],
                device_id=(src,),
                device_id_type=pl.DeviceIdType.MESH,
            ).wait_recv()

        out_ref[:] = gather_ref[:].astype(jnp.float32).reshape(B, SQ, d_model)

        for j in range(1, N_DEV):
            tgt = (my + j) % N_DEV
            pltpu.make_async_remote_copy(
                src_ref=accum_ref.at[pl.ds(tgt * ch, ch)],
                dst_ref=recv1_ref.at[my],
                send_sem=send1_sems.at[tgt],
                recv_sem=recv1_sems.at[my],
                device_id=(tgt,),
                device_id_type=pl.DeviceIdType.MESH,
            ).wait_send()
            pltpu.make_async_remote_copy(
                src_ref=gather_ref.at[pl.ds(my * ch, ch)],
                dst_ref=gather_ref.at[pl.ds(my * ch, ch)],
                send_sem=send2_sems.at[tgt],
                recv_sem=recv2_sems.at[my],
                device_id=(tgt,),
                device_id_type=pl.DeviceIdType.MESH,
            ).wait_send()

    return pl.pallas_call(
        body,
        out_shape=jax.ShapeDtypeStruct((B, SQ, d_model), jnp.float32),
        in_specs=[
            pl.BlockSpec(memory_space=pltpu.VMEM),
            pl.BlockSpec(memory_space=pltpu.VMEM),
            pl.BlockSpec(memory_space=pltpu.VMEM),
            pl.BlockSpec(memory_space=pltpu.VMEM),
            pl.BlockSpec(memory_space=pltpu.VMEM),
        ],
        out_specs=pl.BlockSpec(memory_space=pltpu.VMEM),
        scratch_shapes=[
            pltpu.VMEM((rows, cols), jnp.bfloat16),
            pltpu.VMEM((rows, d_model), jnp.bfloat16),
            pltpu.VMEM((N_DEV, ch, d_model), jnp.bfloat16),
            pltpu.VMEM((rows, d_model), jnp.bfloat16),
            pltpu.SemaphoreType.DMA((N_DEV,)),
            pltpu.SemaphoreType.DMA((N_DEV,)),
            pltpu.SemaphoreType.DMA((N_DEV,)),
            pltpu.SemaphoreType.DMA((N_DEV,)),
        ],
        compiler_params=pltpu.CompilerParams(collective_id=0),
    )(x, Wq_s, K_t, V_t, Wo_s)


# device time: 27513 ns/iter; 1.0644x vs baseline; 1.0043x over previous
import jax
import jax.numpy as jnp
from jax import lax
from jax.experimental import pallas as pl
from jax.experimental.pallas import tpu as pltpu

N_DEV = 32

B = 2
SQ = 128
SKV = 128
DH = 64


def kernel(x, Wq, K_ext, V_ext, Wo):
    hq_per = K_ext.shape[2]
    cols = hq_per * DH
    d_model = x.shape[-1]
    rows = B * SQ
    ch = rows // N_DEV

    idx = lax.axis_index("i")
    Wq_s = lax.dynamic_slice_in_dim(Wq, idx * cols, cols, axis=1).astype(
        jnp.bfloat16
    )
    Wo_s = lax.dynamic_slice_in_dim(Wo, idx * cols, cols, axis=0).astype(
        jnp.bfloat16
    )
    K_t = K_ext.reshape(B, SKV, hq_per * DH)
    V_t = V_ext.reshape(B, SKV, hq_per * DH)

    def body(x_ref, wq_ref, k_ref, v_ref, wo_ref, out_ref,
             ctx_ref, accum_ref, recv1_ref, gather_ref,
             send1_sems, recv1_sems, send2_sems, recv2_sems):
        my = lax.axis_index("i")

        barrier = pltpu.get_barrier_semaphore()
        for j in range(1, N_DEV):
            tgt = (my + j) % N_DEV
            pl.semaphore_signal(
                barrier, inc=1,
                device_id=(tgt,), device_id_type=pl.DeviceIdType.MESH,
            )
        pl.semaphore_wait(barrier, N_DEV - 1)

        x2 = x_ref[:].reshape(rows, d_model).astype(jnp.bfloat16)
        q_all = (
            jnp.dot(x2, wq_ref[:], preferred_element_type=jnp.float32)
            * 0.125
        ).astype(jnp.bfloat16)

        pairs = [(b, h) for b in range(B) for h in range(hq_per)]
        s_all = jnp.concatenate(
            [
                lax.dot_general(
                    q_all[b * SQ:(b + 1) * SQ, h * DH:(h + 1) * DH],
                    k_ref[b, :, h * DH:(h + 1) * DH].astype(jnp.bfloat16),
                    (((1,), (1,)), ((), ())),
                    preferred_element_type=jnp.float32,
                )
                for (b, h) in pairs
            ],
            axis=0,
        )
        n_stack = len(pairs) * SQ
        ii = (lax.broadcasted_iota(jnp.int32, (n_stack, SKV), 0) % SQ) // 64
        jj = lax.broadcasted_iota(jnp.int32, (n_stack, SKV), 1) // 64
        mask = (ii == jj) | (jj == 0) | ((ii + jj) % 3 == 0)
        s_all = jnp.where(mask, s_all, -1e9)
        w_all = jnp.exp(s_all)
        w_all = (w_all / jnp.sum(w_all, axis=1, keepdims=True)).astype(
            jnp.bfloat16
        )
        for p, (b, h) in enumerate(pairs):
            ctx_ref[b * SQ:(b + 1) * SQ, h * DH:(h + 1) * DH] = jnp.dot(
                w_all[p * SQ:(p + 1) * SQ, :],
                v_ref[b, :, h * DH:(h + 1) * DH].astype(jnp.bfloat16),
                preferred_element_type=jnp.float32,
            ).astype(jnp.bfloat16)

        blk_rows = 64
        for blk in range(rows // blk_rows):
            accum_ref[blk * blk_rows:(blk + 1) * blk_rows, :] = jnp.dot(
                ctx_ref[blk * blk_rows:(blk + 1) * blk_rows, :], wo_ref[:],
                preferred_element_type=jnp.float32,
            ).astype(jnp.bfloat16)
            for jj in range(blk_rows // ch):
                c = blk * (blk_rows // ch) + (my + jj) % (blk_rows // ch)
                rdma = pltpu.make_async_remote_copy(
                    src_ref=accum_ref.at[pl.ds(c * ch, ch)],
                    dst_ref=recv1_ref.at[my],
                    send_sem=send1_sems.at[c],
                    recv_sem=recv1_sems.at[my],
                    device_id=(c,),
                    device_id_type=pl.DeviceIdType.MESH,
                )

                @pl.when(c != my)
                def _():
                    rdma.start()

        recv1_ref[pl.ds(my, 1)] = accum_ref[pl.ds(my * ch, ch)].reshape(
            1, ch, d_model
        )

        for j in range(1, N_DEV):
            src = (my + j) % N_DEV
            pltpu.make_async_remote_copy(
                src_ref=accum_ref.at[pl.ds(0, ch)],
                dst_ref=recv1_ref.at[src],
                send_sem=send1_sems.at[0],
                recv_sem=recv1_sems.at[src],
                device_id=(src,),
                device_id_type=pl.DeviceIdType.MESH,
            ).wait_recv()

        red = jnp.sum(
            recv1_ref[:].astype(jnp.float32), axis=0
        ).astype(jnp.bfloat16)
        gather_ref[pl.ds(my * ch, ch)] = red

        for j in range(1, N_DEV):
            tgt = (my + j) % N_DEV
            rdma = pltpu.make_async_remote_copy(
                src_ref=gather_ref.at[pl.ds(my * ch, ch)],
                dst_ref=gather_ref.at[pl.ds(my * ch, ch)],
                send_sem=send2_sems.at[tgt],
                recv_sem=recv2_sems.at[my],
                device_id=(tgt,),
                device_id_type=pl.DeviceIdType.MESH,
            )
            rdma.start()

        for j in range(1, N_DEV):
            src = (my + j) % N_DEV
            pltpu.make_async_remote_copy(
                src_ref=gather_ref.at[pl.ds(0, ch)],
                dst_ref=gather_ref.at[pl.ds(src * ch, ch)],
                send_sem=send2_sems.at[0],
                recv_sem=recv2_sems.at[src],
                device_id=(src,),
                device_id_type=pl.DeviceIdType.MESH,
            ).wait_recv()

        out_ref[:] = gather_ref[:].astype(jnp.float32).reshape(B, SQ, d_model)

        for j in range(1, N_DEV):
            tgt = (my + j) % N_DEV
            pltpu.make_async_remote_copy(
                src_ref=accum_ref.at[pl.ds(tgt * ch, ch)],
                dst_ref=recv1_ref.at[my],
                send_sem=send1_sems.at[tgt],
                recv_sem=recv1_sems.at[my],
                device_id=(tgt,),
                device_id_type=pl.DeviceIdType.MESH,
            ).wait_send()
            pltpu.make_async_remote_copy(
                src_ref=gather_ref.at[pl.ds(my * ch, ch)],
                dst_ref=gather_ref.at[pl.ds(my * ch, ch)],
                send_sem=send2_sems.at[tgt],
                recv_sem=recv2_sems.at[my],
                device_id=(tgt,),
                device_id_type=pl.DeviceIdType.MESH,
            ).wait_send()

    return pl.pallas_call(
        body,
        out_shape=jax.ShapeDtypeStruct((B, SQ, d_model), jnp.float32),
        in_specs=[
            pl.BlockSpec(memory_space=pltpu.VMEM),
            pl.BlockSpec(memory_space=pltpu.VMEM),
            pl.BlockSpec(memory_space=pltpu.VMEM),
            pl.BlockSpec(memory_space=pltpu.VMEM),
            pl.BlockSpec(memory_space=pltpu.VMEM),
        ],
        out_specs=pl.BlockSpec(memory_space=pltpu.VMEM),
        scratch_shapes=[
            pltpu.VMEM((rows, cols), jnp.bfloat16),
            pltpu.VMEM((rows, d_model), jnp.bfloat16),
            pltpu.VMEM((N_DEV, ch, d_model), jnp.bfloat16),
            pltpu.VMEM((rows, d_model), jnp.bfloat16),
            pltpu.SemaphoreType.DMA((N_DEV,)),
            pltpu.SemaphoreType.DMA((N_DEV,)),
            pltpu.SemaphoreType.DMA((N_DEV,)),
            pltpu.SemaphoreType.DMA((N_DEV,)),
        ],
        compiler_params=pltpu.CompilerParams(collective_id=0),
    )(x, Wq_s, K_t, V_t, Wo_s)
